# Initial kernel scaffold; baseline (speedup 1.0000x reference)
import jax
import jax.numpy as jnp
from jax.experimental import pallas as pl


def kernel(h_output, node_biases, src, dst, s2d, s2dc, d2s, d2sc):
    raise NotImplementedError("write your pallas kernel here")


# speedup vs baseline: 1.6062x; 1.6062x over previous
import numpy as np

import jax
import jax.numpy as jnp
from jax import lax
from jax.experimental import pallas as pl
from jax.experimental.pallas import tpu as pltpu
from jax.experimental.pallas import tpu_sc as plsc

D = 64
L = 50
PAIR = 2 * L
NC, NS = 2, 16
NW = NC * NS
LANES = 16
NBUF = 4
GRP = 8


def _lane_sum(v):
    parts = [v[i] for i in range(LANES)]
    while len(parts) > 1:
        parts = [parts[i] + parts[i + 1] for i in range(0, len(parts), 2)]
    return parts[0]


def _sc_body(table, biases, src, dst, sp1, dp1, s2d_r, d2s_r, s2dc, d2sc,
             out,
             idx_v, hpart_v, coef_v, bias_v, srci_v, dsti_v, sp1_v, dp1_v,
             score_v, row0_v,
             bufA0, bufA1, bufA2, bufA3, bufB0, bufB1, bufB2, bufB3,
             psem, semA0, semA1, semA2, semA3, semB0, semB1, semB2, semB3):
    B = out.shape[0]
    epw = B // NW
    jmax = epw // 2
    wid = lax.axis_index("s") * NC + lax.axis_index("c")
    e0 = wid * epw

    bufsA = [bufA0, bufA1, bufA2, bufA3]
    bufsB = [bufB0, bufB1, bufB2, bufB3]
    semsA = [semA0, semA1, semA2, semA3]
    semsB = [semB0, semB1, semB2, semB3]

    pltpu.sync_copy(src.at[pl.ds(e0, epw)], srci_v)
    pltpu.sync_copy(dst.at[pl.ds(e0, epw)], dsti_v)
    pltpu.sync_copy(sp1.at[pl.ds(e0, epw)], sp1_v)
    pltpu.sync_copy(dp1.at[pl.ds(e0, epw)], dp1_v)
    pltpu.sync_copy(s2d_r.at[pl.ds(wid * jmax, jmax)], idx_v.at[pl.ds(0, jmax)])
    pltpu.sync_copy(d2s_r.at[pl.ds(wid * jmax, jmax)], idx_v.at[pl.ds(jmax, jmax)])
    pltpu.sync_copy(s2dc.at[pl.ds(e0, epw)], coef_v.at[pl.ds(0, epw)])
    pltpu.sync_copy(d2sc.at[pl.ds(e0, epw)], coef_v.at[pl.ds(epw, epw)])
    pltpu.sync_copy(table.at[pl.ds(0, 1)], row0_v)

    prologue = [
        pltpu.make_async_copy(table.at[dsti_v], hpart_v.at[pl.ds(0, epw)], psem),
        pltpu.make_async_copy(table.at[srci_v], hpart_v.at[pl.ds(epw, epw)], psem),
        pltpu.make_async_copy(biases.at[sp1_v], bias_v.at[pl.ds(0, epw)], psem),
        pltpu.make_async_copy(biases.at[dp1_v], bias_v.at[pl.ds(epw, epw)], psem),
    ]
    for cp in prologue:
        cp.start()

    def pair_copies(j, p):
        a = pltpu.make_async_copy(table.at[idx_v.at[j]], bufsA[p], semsA[p])
        b = pltpu.make_async_copy(table.at[idx_v.at[jmax + j]], bufsB[p], semsB[p])
        return a, b

    for j0 in range(NBUF - 1):
        a, b = pair_copies(j0, j0)
        a.start()
        b.start()

    for cp in prologue:
        cp.wait()

    iota = lax.iota(jnp.int32, LANES)
    one = jnp.ones((LANES,), jnp.int32)
    m_lt2 = jnp.maximum(0, jnp.minimum(1, 2 - iota))
    m_ge2 = one - m_lt2
    m_ge12 = jnp.maximum(0, jnp.minimum(1, iota - 11))

    def zero_count_vecs(rowi):
        def zc(off):
            v = idx_v[rowi, pl.ds(off, LANES)]
            return one - jnp.minimum(jnp.abs(v), 1)

        z48 = zc(48)
        v_left = zc(0) + zc(16) + zc(32) + m_lt2 * z48
        v_right = m_ge2 * z48 + zc(64) + zc(80) + m_ge12 * zc(84)
        return v_left, v_right

    r0 = [row0_v[0, pl.ds(c * LANES, LANES)] for c in range(4)]
    onehots = [(one - jnp.minimum(jnp.abs(iota - i), 1)).astype(jnp.float32)
               for i in range(LANES)]

    def outer(g, carry):
        gb = g * LANES
        c1v = coef_v[pl.ds(gb, LANES)]
        c2v = coef_v[pl.ds(epw + gb, LANES)]
        c1sqv = c1v * c1v
        c2sqv = c2v * c2v
        csq1 = [c1sqv[i] for i in range(LANES)]
        csq2 = [c2sqv[i] for i in range(LANES)]
        sv = jnp.zeros((LANES,), jnp.float32)

        for k in range(GRP):
            j = g * GRP + k
            p = k % NBUF

            @pl.when(j + NBUF - 1 < jmax)
            def _():
                a2, b2 = pair_copies(j + NBUF - 1, (k + NBUF - 1) % NBUF)
                a2.start()
                b2.start()

            a, b = pair_copies(j, p)
            a.wait()
            b.wait()

            bA, bB = bufsA[p], bufsB[p]
            vlA, vrA = zero_count_vecs(j)
            vlB, vrB = zero_count_vecs(jmax + j)
            packed = (vlA + (vrA << 6)) + ((vlB << 12) + (vrB << 18))
            tot = _lane_sum(packed)
            nA = (tot & 63, (tot >> 6) & 63)
            nB = ((tot >> 12) & 63, (tot >> 18) & 63)

            for e01 in range(2):
                e = 2 * j + e01
                rbase = e01 * L

                def row(l, accs):
                    r = rbase + l
                    new = []
                    for c in range(4):
                        sl = pl.ds(c * LANES, LANES)
                        new.append(accs[c] + bA[r, sl])
                    for c in range(4):
                        sl = pl.ds(c * LANES, LANES)
                        new.append(accs[4 + c] + bB[r, sl])
                    return tuple(new)

                zeros = tuple(jnp.zeros((LANES,), jnp.float32)
                              for _ in range(8))
                accs = lax.fori_loop(0, L, row, zeros, unroll=5)

                naf = nA[e01].astype(jnp.float32)
                nbf = nB[e01].astype(jnp.float32)
                q = 2 * k + e01
                w = jnp.zeros((LANES,), jnp.float32)
                for c in range(4):
                    sl = pl.ds(c * LANES, LANES)
                    hd = hpart_v[e, sl]
                    hs = hpart_v[epw + e, sl]
                    accA = accs[c] - naf * r0[c]
                    accB = accs[4 + c] - nbf * r0[c]
                    w = w + hd * (hs + csq1[q] * accA) + (csq2[q] * hs) * accB
                sv = sv + onehots[q] * _lane_sum(w)

        sv = sv + bias_v[pl.ds(gb, LANES)] + bias_v[pl.ds(epw + gb, LANES)]
        score_v[pl.ds(gb, LANES)] = sv
        return carry

    lax.fori_loop(0, jmax // GRP, outer, 0)

    pltpu.sync_copy(score_v, out.at[pl.ds(e0, epw)])


@jax.jit
def kernel(h_output, node_biases, src, dst, s2d, s2dc, d2s, d2sc):
    B, Lx = s2d.shape
    assert Lx == L and h_output.shape[1] == D and B % (2 * NW * GRP) == 0
    epw = B // NW

    s2d_r = s2d.reshape(B * L // PAIR, PAIR)
    d2s_r = d2s.reshape(B * L // PAIR, PAIR)
    sp1 = src + 1
    dp1 = dst + 1

    mesh = plsc.VectorSubcoreMesh(core_axis_name="c", subcore_axis_name="s",
                                  num_cores=NC, num_subcores=NS)
    f = pl.kernel(
        _sc_body,
        out_type=jax.ShapeDtypeStruct((B,), jnp.float32),
        mesh=mesh,
        compiler_params=pltpu.CompilerParams(use_tc_tiling_on_sc=False),
        scratch_types=[
            pltpu.VMEM((2 * (epw // 2), PAIR), jnp.int32),
            pltpu.VMEM((2 * epw, D), jnp.float32),
            pltpu.VMEM((2 * epw,), jnp.float32),
            pltpu.VMEM((2 * epw,), jnp.float32),
            pltpu.VMEM((epw,), jnp.int32),
            pltpu.VMEM((epw,), jnp.int32),
            pltpu.VMEM((epw,), jnp.int32),
            pltpu.VMEM((epw,), jnp.int32),
            pltpu.VMEM((epw,), jnp.float32),
            pltpu.VMEM((1, D), jnp.float32),
            pltpu.VMEM((PAIR, D), jnp.float32),
            pltpu.VMEM((PAIR, D), jnp.float32),
            pltpu.VMEM((PAIR, D), jnp.float32),
            pltpu.VMEM((PAIR, D), jnp.float32),
            pltpu.VMEM((PAIR, D), jnp.float32),
            pltpu.VMEM((PAIR, D), jnp.float32),
            pltpu.VMEM((PAIR, D), jnp.float32),
            pltpu.VMEM((PAIR, D), jnp.float32),
            pltpu.SemaphoreType.DMA,
            pltpu.SemaphoreType.DMA,
            pltpu.SemaphoreType.DMA,
            pltpu.SemaphoreType.DMA,
            pltpu.SemaphoreType.DMA,
            pltpu.SemaphoreType.DMA,
            pltpu.SemaphoreType.DMA,
            pltpu.SemaphoreType.DMA,
            pltpu.SemaphoreType.DMA,
        ],
    )
    return f(h_output, node_biases, src, dst, sp1, dp1, s2d_r, d2s_r,
             s2dc, d2sc)
